# baseline (speedup 1.0000x reference)
import functools

import jax
import jax.numpy as jnp
from jax.experimental import pallas as pl

N_LABELS = 4
IN_SIZE = 128
OUT_SIZE = 128
BLK = 1280


def _ggnn_body(e_ref, x_ref, wt_ref, o_ref):
    x = x_ref[...].astype(jnp.bfloat16)
    p = jnp.dot(x, wt_ref[...].astype(jnp.bfloat16),
                preferred_element_type=jnp.float32)
    e = e_ref[...]
    acc = jnp.where(e == 0, p[:, 0:OUT_SIZE], 0.0)
    for lab in range(1, N_LABELS):
        acc = acc + jnp.where(e == lab, p[:, lab * OUT_SIZE:(lab + 1) * OUT_SIZE], 0.0)
    o_ref[...] = acc


@functools.partial(jax.jit, static_argnames=("interpret",))
def kernel(h_v, h_w, e_vw, edge_matrix, interpret=False):
    del h_v
    b, n, _ = h_w.shape
    rows = b * n
    x = h_w.reshape(rows, IN_SIZE)
    e = e_vw.reshape(rows, 1)
    wt = jnp.transpose(edge_matrix, (2, 0, 1)).reshape(IN_SIZE, N_LABELS * OUT_SIZE)

    grid = rows // BLK
    out = pl.pallas_call(
        _ggnn_body,
        grid=(grid,),
        in_specs=[
            pl.BlockSpec((BLK, 1), lambda i: (i, 0)),
            pl.BlockSpec((BLK, IN_SIZE), lambda i: (i, 0)),
            pl.BlockSpec((IN_SIZE, N_LABELS * OUT_SIZE), lambda i: (0, 0)),
        ],
        out_specs=pl.BlockSpec((BLK, OUT_SIZE), lambda i: (i, 0)),
        out_shape=jax.ShapeDtypeStruct((rows, OUT_SIZE), h_w.dtype),
        interpret=interpret,
    )(e, x, wt)
    return out.reshape(b, n, OUT_SIZE)


# speedup vs baseline: 1.1055x; 1.1055x over previous
import functools

import jax
import jax.numpy as jnp
from jax.experimental import pallas as pl

N_LABELS = 4
IN_SIZE = 128
OUT_SIZE = 128
BLK = 1280


def _ggnn_body(e_ref, x_ref, ws_ref, o_ref):
    x = x_ref[...].astype(jnp.bfloat16)
    ebc = jnp.broadcast_to(e_ref[...], (BLK, IN_SIZE))
    zero = jnp.zeros((), jnp.bfloat16)
    xcat = jnp.concatenate(
        [jnp.where(ebc == jnp.bfloat16(lab), x, zero) for lab in range(N_LABELS)],
        axis=1)
    o_ref[...] = jnp.dot(xcat, ws_ref[...], preferred_element_type=jnp.float32)


@functools.partial(jax.jit, static_argnames=("interpret",))
def kernel(h_v, h_w, e_vw, edge_matrix, interpret=False):
    del h_v
    b, n, _ = h_w.shape
    rows = b * n
    x = h_w.reshape(rows, IN_SIZE)
    eb = e_vw.reshape(rows, 1).astype(jnp.bfloat16)
    ws = jnp.transpose(edge_matrix, (0, 2, 1)).reshape(N_LABELS * IN_SIZE, OUT_SIZE)
    ws = ws.astype(jnp.bfloat16)

    grid = rows // BLK
    out = pl.pallas_call(
        _ggnn_body,
        grid=(grid,),
        in_specs=[
            pl.BlockSpec((BLK, 1), lambda i: (i, 0)),
            pl.BlockSpec((BLK, IN_SIZE), lambda i: (i, 0)),
            pl.BlockSpec((N_LABELS * IN_SIZE, OUT_SIZE), lambda i: (0, 0)),
        ],
        out_specs=pl.BlockSpec((BLK, OUT_SIZE), lambda i: (i, 0)),
        out_shape=jax.ShapeDtypeStruct((rows, OUT_SIZE), h_w.dtype),
        interpret=interpret,
    )(eb, x, ws)
    return out.reshape(b, n, OUT_SIZE)


# speedup vs baseline: 2.0444x; 1.8493x over previous
import functools

import jax
import jax.numpy as jnp
from jax.experimental import pallas as pl

N_LABELS = 4
IN_SIZE = 128
OUT_SIZE = 128
BB = 200
N = 32


def _ggnn_body(e_ref, x_ref, ws_ref, o_ref):
    rows = BB * N
    x = x_ref[...].reshape(rows, IN_SIZE).astype(jnp.bfloat16)
    e = e_ref[...].reshape(rows, 1).astype(jnp.bfloat16)
    ebc = jnp.broadcast_to(e, (rows, IN_SIZE))
    zero = jnp.zeros((), jnp.bfloat16)
    xcat = jnp.concatenate(
        [jnp.where(ebc == jnp.bfloat16(lab), x, zero) for lab in range(N_LABELS)],
        axis=1)
    p = jnp.dot(xcat, ws_ref[...], preferred_element_type=jnp.float32)
    o_ref[...] = p.reshape(BB, N, OUT_SIZE)


@functools.partial(jax.jit, static_argnames=("interpret",))
def kernel(h_v, h_w, e_vw, edge_matrix, interpret=False):
    del h_v
    b, n, _ = h_w.shape
    ws = jnp.transpose(edge_matrix, (0, 2, 1)).reshape(N_LABELS * IN_SIZE, OUT_SIZE)
    ws = ws.astype(jnp.bfloat16)

    grid = b // BB
    out = pl.pallas_call(
        _ggnn_body,
        grid=(grid,),
        in_specs=[
            pl.BlockSpec((BB, N, 1), lambda i: (i, 0, 0)),
            pl.BlockSpec((BB, N, IN_SIZE), lambda i: (i, 0, 0)),
            pl.BlockSpec((N_LABELS * IN_SIZE, OUT_SIZE), lambda i: (0, 0)),
        ],
        out_specs=pl.BlockSpec((BB, N, OUT_SIZE), lambda i: (i, 0, 0)),
        out_shape=jax.ShapeDtypeStruct((b, n, OUT_SIZE), h_w.dtype),
        interpret=interpret,
    )(e_vw, h_w, ws)
    return out


# speedup vs baseline: 3.5618x; 1.7422x over previous
import functools

import jax
import jax.numpy as jnp
from jax.experimental import pallas as pl

N_LABELS = 4
IN_SIZE = 128
OUT_SIZE = 128
BB = 200
N = 32


def _ggnn_body(e_ref, x_ref, ws_ref, o_ref):
    rows = BB * N
    x = x_ref[...].reshape(rows, IN_SIZE).astype(jnp.bfloat16)
    e = e_ref[...].astype(jnp.bfloat16)
    ebc = jnp.broadcast_to(e[:, :, None], (BB, N, IN_SIZE)).reshape(rows, IN_SIZE)
    zero = jnp.zeros((), jnp.bfloat16)
    xcat = jnp.concatenate(
        [jnp.where(ebc == jnp.bfloat16(lab), x, zero) for lab in range(N_LABELS)],
        axis=1)
    p = jnp.dot(xcat, ws_ref[...], preferred_element_type=jnp.float32)
    o_ref[...] = p.reshape(BB, N, OUT_SIZE)


@functools.partial(jax.jit, static_argnames=("interpret",))
def kernel(h_v, h_w, e_vw, edge_matrix, interpret=False):
    del h_v
    b, n, _ = h_w.shape
    ws = jnp.transpose(edge_matrix, (0, 2, 1)).reshape(N_LABELS * IN_SIZE, OUT_SIZE)
    ws = ws.astype(jnp.bfloat16)

    e2 = e_vw.reshape(b, n)

    grid = b // BB
    out = pl.pallas_call(
        _ggnn_body,
        grid=(grid,),
        in_specs=[
            pl.BlockSpec((BB, N), lambda i: (i, 0)),
            pl.BlockSpec((BB, N, IN_SIZE), lambda i: (i, 0, 0)),
            pl.BlockSpec((N_LABELS * IN_SIZE, OUT_SIZE), lambda i: (0, 0)),
        ],
        out_specs=pl.BlockSpec((BB, N, OUT_SIZE), lambda i: (i, 0, 0)),
        out_shape=jax.ShapeDtypeStruct((b, n, OUT_SIZE), h_w.dtype),
        interpret=interpret,
    )(e2, h_w, ws)
    return out


# speedup vs baseline: 3.9922x; 1.1208x over previous
import functools

import jax
import jax.numpy as jnp
from jax.experimental import pallas as pl

N_LABELS = 4
IN_SIZE = 128
OUT_SIZE = 128
BB = 400
N = 32


def _ggnn_body(e_ref, x_ref, ws_ref, o_ref):
    rows = BB * N
    x = x_ref[...].reshape(rows, IN_SIZE).astype(jnp.bfloat16)
    e = e_ref[...].astype(jnp.bfloat16)
    ebc = jnp.broadcast_to(e[:, :, None], (BB, N, IN_SIZE)).reshape(rows, IN_SIZE)
    zero = jnp.zeros((), jnp.bfloat16)
    xcat = jnp.concatenate(
        [jnp.where(ebc == jnp.bfloat16(lab), x, zero) for lab in range(N_LABELS)],
        axis=1)
    p = jnp.dot(xcat, ws_ref[...], preferred_element_type=jnp.float32)
    o_ref[...] = p.reshape(BB, N, OUT_SIZE)


@functools.partial(jax.jit, static_argnames=("interpret",))
def kernel(h_v, h_w, e_vw, edge_matrix, interpret=False):
    del h_v
    b, n, _ = h_w.shape
    ws = jnp.transpose(edge_matrix, (0, 2, 1)).reshape(N_LABELS * IN_SIZE, OUT_SIZE)
    ws = ws.astype(jnp.bfloat16)

    e2 = e_vw.reshape(b, n)

    grid = b // BB
    out = pl.pallas_call(
        _ggnn_body,
        grid=(grid,),
        in_specs=[
            pl.BlockSpec((BB, N), lambda i: (i, 0)),
            pl.BlockSpec((BB, N, IN_SIZE), lambda i: (i, 0, 0)),
            pl.BlockSpec((N_LABELS * IN_SIZE, OUT_SIZE), lambda i: (0, 0)),
        ],
        out_specs=pl.BlockSpec((BB, N, OUT_SIZE), lambda i: (i, 0, 0)),
        out_shape=jax.ShapeDtypeStruct((b, n, OUT_SIZE), h_w.dtype),
        interpret=interpret,
    )(e2, h_w, ws)
    return out


# speedup vs baseline: 3.9945x; 1.0006x over previous
import functools

import jax
import jax.numpy as jnp
from jax.experimental import pallas as pl
from jax.experimental.pallas import tpu as pltpu

N_LABELS = 4
IN_SIZE = 128
OUT_SIZE = 128
BB = 400
N = 32


def _ggnn_body(e_ref, x_ref, ws_ref, o_ref):
    rows = BB * N
    x = x_ref[...].reshape(rows, IN_SIZE).astype(jnp.bfloat16)
    e = e_ref[...].astype(jnp.bfloat16)
    ebc = jnp.broadcast_to(e[:, :, None], (BB, N, IN_SIZE)).reshape(rows, IN_SIZE)
    zero = jnp.zeros((), jnp.bfloat16)
    xcat = jnp.concatenate(
        [jnp.where(ebc == jnp.bfloat16(lab), x, zero) for lab in range(N_LABELS)],
        axis=1)
    p = jnp.dot(xcat, ws_ref[...], preferred_element_type=jnp.float32)
    o_ref[...] = p.reshape(BB, N, OUT_SIZE)


@functools.partial(jax.jit, static_argnames=("interpret",))
def kernel(h_v, h_w, e_vw, edge_matrix, interpret=False):
    del h_v
    b, n, _ = h_w.shape
    ws = jnp.transpose(edge_matrix, (0, 2, 1)).reshape(N_LABELS * IN_SIZE, OUT_SIZE)
    ws = ws.astype(jnp.bfloat16)

    e2 = e_vw.reshape(b, n)

    grid = b // BB
    out = pl.pallas_call(
        _ggnn_body,
        grid=(grid,),
        in_specs=[
            pl.BlockSpec((BB, N), lambda i: (i, 0)),
            pl.BlockSpec((BB, N, IN_SIZE), lambda i: (i, 0, 0)),
            pl.BlockSpec((N_LABELS * IN_SIZE, OUT_SIZE), lambda i: (0, 0)),
        ],
        out_specs=pl.BlockSpec((BB, N, OUT_SIZE), lambda i: (i, 0, 0)),
        out_shape=jax.ShapeDtypeStruct((b, n, OUT_SIZE), h_w.dtype),
        interpret=interpret,
        compiler_params=pltpu.CompilerParams(dimension_semantics=("parallel",)),
    )(e2, h_w, ws)
    return out
